# baseline (speedup 1.0000x reference)
import functools

import jax
import jax.numpy as jnp
from jax import lax
from jax.experimental import pallas as pl
from jax.experimental.pallas import tpu as pltpu
from jax.experimental.pallas import tpu_sc as plsc

N = 10000
E = 320000
D = 128

NC = 2
NS = 16
NW = NC * NS

EB = 64
NB = 160
EW = NB * EB
EP = NW * EW
PH = 8
NPH = NB // PH

NP = 10112
RPT = NP // NS


def _sc_agg_build():
    mesh = plsc.VectorSubcoreMesh(core_axis_name="c", subcore_axis_name="s")

    @functools.partial(
        pl.kernel,
        out_type=jax.ShapeDtypeStruct((NC * NP, D), jnp.float32),
        mesh=mesh,
        scratch_types=[
            pltpu.VMEM((EW,), jnp.int32),
            pltpu.VMEM((PH, EB), jnp.int32),
            pltpu.VMEM((PH, EB), jnp.int32),
            pltpu.VMEM((EB, D), jnp.float32),
            pltpu.VMEM((EB, D), jnp.float32),
            pltpu.VMEM((EB, D), jnp.float32),
            pltpu.VMEM((EB, D), jnp.float32),
            pltpu.VMEM_SHARED((NP, D), jnp.float32),
            pltpu.SemaphoreType.DMA,
            pltpu.SemaphoreType.DMA,
            pltpu.SemaphoreType.DMA,
            pltpu.SemaphoreType.DMA,
            pltpu.SemaphoreType.DMA,
            pltpu.SemaphoreType.DMA,
        ],
    )
    def sc_agg(x_hbm, src_hbm, dst2_hbm, zeros_hbm, out_hbm,
               src_v, dstb0, dstb1, rows0, rows1, rows2, rows3, acc,
               sg0, sg1, sg2, sg3, si0, si1):
        c = lax.axis_index("c")
        s = lax.axis_index("s")
        w = c * NS + s
        r0 = w * NB

        rows = (rows0, rows1, rows2, rows3)
        sg = (sg0, sg1, sg2, sg3)

        XR = N - 15 * RPT

        @pl.when(c == 1)
        def _():
            pltpu.sync_copy(zeros_hbm.at[pl.ds(s * RPT, RPT)],
                            acc.at[pl.ds(s * RPT, RPT)])

        @pl.when(jnp.logical_and(c == 0, s < NS - 1))
        def _():
            pltpu.sync_copy(x_hbm.at[pl.ds(s * RPT, RPT)],
                            acc.at[pl.ds(s * RPT, RPT)])

        @pl.when(jnp.logical_and(c == 0, s == NS - 1))
        def _():
            pltpu.sync_copy(x_hbm.at[pl.ds((NS - 1) * RPT, XR)],
                            acc.at[pl.ds((NS - 1) * RPT, XR)])
            pltpu.sync_copy(zeros_hbm.at[pl.ds(N, NP - N)],
                            acc.at[pl.ds(N, NP - N)])
        pltpu.sync_copy(src_hbm.at[pl.ds(w * EW, EW)], src_v)
        pltpu.sync_copy(dst2_hbm.at[pl.ds(r0, PH)], dstb0)
        pltpu.async_copy(dst2_hbm.at[pl.ds(r0 + PH, PH)], dstb1, si1)
        pltpu.async_copy(x_hbm.at[src_v.at[pl.ds(0, EB)]], rows0, sg0)
        pltpu.async_copy(x_hbm.at[src_v.at[pl.ds(EB, EB)]], rows1, sg1)
        pltpu.async_copy(x_hbm.at[src_v.at[pl.ds(2 * EB, EB)]], rows2, sg2)
        plsc.subcore_barrier()

        def step(b_dyn, j, dstb, jj):
            k = j & 3
            pltpu.make_async_copy(
                x_hbm.at[src_v.at[pl.ds(0, EB)]], rows[k], sg[k]).wait()
            nxt = (b_dyn + (j + 3)) * EB
            kn = (j + 3) & 3

            def start_next():
                pltpu.async_copy(
                    x_hbm.at[src_v.at[pl.ds(nxt, EB)]], rows[kn], sg[kn])
            if j < 13:
                start_next()
            else:
                pl.when(b_dyn + j + 3 < NB)(start_next)
            pltpu.sync_copy(rows[k], acc.at[dstb.at[jj]], add=True)

        def body(i, carry):
            b_dyn = i * 2 * PH

            @pl.when(i > 0)
            def _():
                pltpu.make_async_copy(
                    dst2_hbm.at[pl.ds(r0, PH)], dstb0, si0).wait()
            for j in range(PH):
                step(b_dyn, j, dstb0, j)
            pltpu.make_async_copy(
                dst2_hbm.at[pl.ds(r0, PH)], dstb1, si1).wait()

            @pl.when(i < NPH // 2 - 1)
            def _():
                pltpu.async_copy(
                    dst2_hbm.at[pl.ds(r0 + (i * 2 + 2) * PH, PH)], dstb0, si0)
            for j in range(PH):
                step(b_dyn, PH + j, dstb1, j)

            @pl.when(i < NPH // 2 - 1)
            def _():
                pltpu.async_copy(
                    dst2_hbm.at[pl.ds(r0 + (i * 2 + 3) * PH, PH)], dstb1, si1)
            return carry

        lax.fori_loop(0, NPH // 2, body, 0, unroll=False)

        plsc.subcore_barrier()
        pltpu.sync_copy(acc.at[pl.ds(s * RPT, RPT)],
                        out_hbm.at[pl.ds(c * NP + s * RPT, RPT)])

    return sc_agg


_sc_agg = _sc_agg_build()


def _mlp_kernel(relu_out, trans_out, p0_ref, p1_ref, wa_ref, ba_ref, wb_ref,
                bb_ref, o_ref):
    rst = p0_ref[...] + p1_ref[...]
    hid = jnp.dot(rst, wa_ref[...], preferred_element_type=jnp.float32)
    hid = jnp.maximum(hid + ba_ref[...], 0.0)
    out = jnp.dot(hid, wb_ref[...], preferred_element_type=jnp.float32)
    out = out + bb_ref[...]
    if relu_out:
        out = jnp.maximum(out, 0.0)
    if trans_out:
        o_ref[...] = out.T
    else:
        o_ref[...] = out


def _mlp(p0, p1, Wa, ba, Wb, bb, relu_out, trans_out, bn=1000):
    if trans_out:
        bn = N
    grid = N // bn
    if trans_out:
        out_spec = pl.BlockSpec((D, N), lambda i: (0, 0))
        out_shape = jax.ShapeDtypeStruct((D, N), jnp.float32)
    else:
        out_spec = pl.BlockSpec((bn, D), lambda i: (i, 0))
        out_shape = jax.ShapeDtypeStruct((N, D), jnp.float32)
    return pl.pallas_call(
        functools.partial(_mlp_kernel, relu_out, trans_out),
        grid=(grid,),
        in_specs=[
            pl.BlockSpec((bn, D), lambda i: (i, 0)),
            pl.BlockSpec((bn, D), lambda i: (i, 0)),
            pl.BlockSpec((D, D), lambda i: (0, 0)),
            pl.BlockSpec((1, D), lambda i: (0, 0)),
            pl.BlockSpec((D, D), lambda i: (0, 0)),
            pl.BlockSpec((1, D), lambda i: (0, 0)),
        ],
        out_specs=out_spec,
        out_shape=out_shape,
    )(p0, p1, Wa, ba.reshape(1, D), Wb, bb.reshape(1, D))


def kernel(h, edge_index, W0a, b0a, W0b, b0b, W1a, b1a, W1b, b1b):
    x = h.T
    pad = EP - E
    pad_ar = jnp.arange(pad, dtype=jnp.int32)
    src = jnp.concatenate([edge_index[0], pad_ar % N])
    dst = jnp.concatenate([edge_index[1], N + pad_ar % (NP - N)])
    dst2 = dst.reshape(EP // EB, EB)

    zeros = jnp.zeros((NP, D), jnp.float32)
    p = _sc_agg(x, src, dst2, zeros)
    x1 = _mlp(p[:N], p[NP:NP + N], W0a, b0a, W0b, b0b,
              relu_out=True, trans_out=False)
    p2 = _sc_agg(x1, src, dst2, zeros)
    return _mlp(p2[:N], p2[NP:NP + N], W1a, b1a, W1b, b1b,
                relu_out=False, trans_out=True)


# speedup vs baseline: 1.0493x; 1.0493x over previous
import functools

import jax
import jax.numpy as jnp
from jax import lax
from jax.experimental import pallas as pl
from jax.experimental.pallas import tpu as pltpu
from jax.experimental.pallas import tpu_sc as plsc

N = 10000
E = 320000
D = 128

NC = 2
NS = 16
NW = NC * NS

EB = 64
NB = 160
EW = NB * EB
EP = NW * EW
PH = 8
NPH = NB // PH

NP = 10112
RPT = NP // NS


def _sc_agg_build():
    mesh = plsc.VectorSubcoreMesh(core_axis_name="c", subcore_axis_name="s")

    @functools.partial(
        pl.kernel,
        out_type=jax.ShapeDtypeStruct((NC * NP, D), jnp.float32),
        mesh=mesh,
        scratch_types=[
            pltpu.VMEM((EW,), jnp.int32),
            pltpu.VMEM((PH, EB), jnp.int32),
            pltpu.VMEM((PH, EB), jnp.int32),
            pltpu.VMEM((EB, D), jnp.float32),
            pltpu.VMEM((EB, D), jnp.float32),
            pltpu.VMEM((EB, D), jnp.float32),
            pltpu.VMEM((EB, D), jnp.float32),
            pltpu.VMEM_SHARED((NP, D), jnp.float32),
            pltpu.SemaphoreType.DMA,
            pltpu.SemaphoreType.DMA,
            pltpu.SemaphoreType.DMA,
            pltpu.SemaphoreType.DMA,
            pltpu.SemaphoreType.DMA,
            pltpu.SemaphoreType.DMA,
        ],
    )
    def sc_agg(x_hbm, src_hbm, dst2_hbm, zeros_hbm, out_hbm,
               src_v, dstb0, dstb1, rows0, rows1, rows2, rows3, acc,
               sg0, sg1, sg2, sg3, si0, si1):
        c = lax.axis_index("c")
        s = lax.axis_index("s")
        w = c * NS + s
        r0 = w * NB

        rows = (rows0, rows1, rows2, rows3)
        sg = (sg0, sg1, sg2, sg3)

        zinit = pltpu.async_copy(zeros_hbm.at[pl.ds(s * RPT, RPT)],
                                 acc.at[pl.ds(s * RPT, RPT)], si0)
        pltpu.sync_copy(src_hbm.at[pl.ds(w * EW, EW)], src_v)
        pltpu.sync_copy(dst2_hbm.at[pl.ds(r0, PH)], dstb0)
        pltpu.async_copy(dst2_hbm.at[pl.ds(r0 + PH, PH)], dstb1, si1)
        pltpu.async_copy(x_hbm.at[src_v.at[pl.ds(0, EB)]], rows0, sg0)
        pltpu.async_copy(x_hbm.at[src_v.at[pl.ds(EB, EB)]], rows1, sg1)
        pltpu.async_copy(x_hbm.at[src_v.at[pl.ds(2 * EB, EB)]], rows2, sg2)
        zinit.wait()
        plsc.subcore_barrier()

        def step(b_dyn, j, dstb, jj):
            k = j & 3
            pltpu.make_async_copy(
                x_hbm.at[src_v.at[pl.ds(0, EB)]], rows[k], sg[k]).wait()
            nxt = (b_dyn + (j + 3)) * EB
            kn = (j + 3) & 3

            def start_next():
                pltpu.async_copy(
                    x_hbm.at[src_v.at[pl.ds(nxt, EB)]], rows[kn], sg[kn])
            if j < 13:
                start_next()
            else:
                pl.when(b_dyn + j + 3 < NB)(start_next)
            pltpu.sync_copy(rows[k], acc.at[dstb.at[jj]], add=True)

        def body(i, carry):
            b_dyn = i * 2 * PH

            @pl.when(i > 0)
            def _():
                pltpu.make_async_copy(
                    dst2_hbm.at[pl.ds(r0, PH)], dstb0, si0).wait()
            for j in range(PH):
                step(b_dyn, j, dstb0, j)
            pltpu.make_async_copy(
                dst2_hbm.at[pl.ds(r0, PH)], dstb1, si1).wait()

            @pl.when(i < NPH // 2 - 1)
            def _():
                pltpu.async_copy(
                    dst2_hbm.at[pl.ds(r0 + (i * 2 + 2) * PH, PH)], dstb0, si0)
            for j in range(PH):
                step(b_dyn, PH + j, dstb1, j)

            @pl.when(i < NPH // 2 - 1)
            def _():
                pltpu.async_copy(
                    dst2_hbm.at[pl.ds(r0 + (i * 2 + 3) * PH, PH)], dstb1, si1)
            return carry

        lax.fori_loop(0, NPH // 2, body, 0, unroll=False)

        plsc.subcore_barrier()
        pltpu.sync_copy(acc.at[pl.ds(s * RPT, RPT)],
                        out_hbm.at[pl.ds(c * NP + s * RPT, RPT)])

    return sc_agg


_sc_agg = _sc_agg_build()


def _mlp_kernel(relu_out, x_ref, p0_ref, p1_ref, wa_ref, ba_ref, wb_ref,
                bb_ref, o_ref):
    rst = x_ref[...] + p0_ref[...] + p1_ref[...]
    hid = jnp.dot(rst, wa_ref[...], preferred_element_type=jnp.float32)
    hid = jnp.maximum(hid + ba_ref[...], 0.0)
    out = jnp.dot(hid, wb_ref[...], preferred_element_type=jnp.float32)
    out = out + bb_ref[...]
    if relu_out:
        out = jnp.maximum(out, 0.0)
    o_ref[...] = out


def _mlp(x, p0, p1, Wa, ba, Wb, bb, relu_out, bn=2000):
    grid = N // bn
    return pl.pallas_call(
        functools.partial(_mlp_kernel, relu_out),
        grid=(grid,),
        in_specs=[
            pl.BlockSpec((bn, D), lambda i: (i, 0)),
            pl.BlockSpec((bn, D), lambda i: (i, 0)),
            pl.BlockSpec((bn, D), lambda i: (i, 0)),
            pl.BlockSpec((D, D), lambda i: (0, 0)),
            pl.BlockSpec((1, D), lambda i: (0, 0)),
            pl.BlockSpec((D, D), lambda i: (0, 0)),
            pl.BlockSpec((1, D), lambda i: (0, 0)),
        ],
        out_specs=pl.BlockSpec((bn, D), lambda i: (i, 0)),
        out_shape=jax.ShapeDtypeStruct((N, D), jnp.float32),
    )(x, p0, p1, Wa, ba.reshape(1, D), Wb, bb.reshape(1, D))


def kernel(h, edge_index, W0a, b0a, W0b, b0b, W1a, b1a, W1b, b1b):
    x = h.T
    pad = EP - E
    pad_ar = jnp.arange(pad, dtype=jnp.int32)
    src = jnp.concatenate([edge_index[0], pad_ar % N])
    dst = jnp.concatenate([edge_index[1], N + pad_ar % (NP - N)])
    dst2 = dst.reshape(EP // EB, EB)

    zeros = jnp.zeros((NP, D), jnp.float32)
    p = _sc_agg(x, src, dst2, zeros)
    x1 = _mlp(x, p[:N], p[NP:NP + N], W0a, b0a, W0b, b0b, relu_out=True)
    p2 = _sc_agg(x1, src, dst2, zeros)
    x2 = _mlp(x1, p2[:N], p2[NP:NP + N], W1a, b1a, W1b, b1b, relu_out=False)
    return x2.T


# speedup vs baseline: 1.0557x; 1.0061x over previous
import functools

import jax
import jax.numpy as jnp
from jax import lax
from jax.experimental import pallas as pl
from jax.experimental.pallas import tpu as pltpu
from jax.experimental.pallas import tpu_sc as plsc

N = 10000
E = 320000
D = 128

NC = 2
NS = 16
NW = NC * NS

EB = 64
NB = 160
EW = NB * EB
EP = NW * EW
PH = 8
NPH = NB // PH

NP = 10112
RPT = NP // NS


def _sc_agg_build():
    mesh = plsc.VectorSubcoreMesh(core_axis_name="c", subcore_axis_name="s")

    @functools.partial(
        pl.kernel,
        out_type=jax.ShapeDtypeStruct((NC * NP, D), jnp.float32),
        mesh=mesh,
        scratch_types=[
            pltpu.VMEM((EW,), jnp.int32),
            pltpu.VMEM((PH, EB), jnp.int32),
            pltpu.VMEM((PH, EB), jnp.int32),
            pltpu.VMEM((EB, D), jnp.float32),
            pltpu.VMEM((EB, D), jnp.float32),
            pltpu.VMEM((EB, D), jnp.float32),
            pltpu.VMEM((EB, D), jnp.float32),
            pltpu.VMEM_SHARED((NP, D), jnp.float32),
            pltpu.SemaphoreType.DMA,
            pltpu.SemaphoreType.DMA,
            pltpu.SemaphoreType.DMA,
            pltpu.SemaphoreType.DMA,
            pltpu.SemaphoreType.DMA,
            pltpu.SemaphoreType.DMA,
        ],
    )
    def sc_agg(x_hbm, src_hbm, dst2_hbm, zeros_hbm, out_hbm,
               src_v, dstb0, dstb1, rows0, rows1, rows2, rows3, acc,
               sg0, sg1, sg2, sg3, si0, si1):
        c = lax.axis_index("c")
        s = lax.axis_index("s")
        w = c * NS + s
        r0 = w * NB

        rows = (rows0, rows1, rows2, rows3)
        sg = (sg0, sg1, sg2, sg3)

        XR = N - (NS - 1) * RPT
        init_src = jnp.where(c == 0, 0, 1)

        @pl.when(c == 1)
        def _():
            pltpu.async_copy(zeros_hbm.at[pl.ds(s * RPT, RPT)],
                             acc.at[pl.ds(s * RPT, RPT)], si0)

        @pl.when(jnp.logical_and(c == 0, s < NS - 1))
        def _():
            pltpu.async_copy(x_hbm.at[pl.ds(s * RPT, RPT)],
                             acc.at[pl.ds(s * RPT, RPT)], si0)

        @pl.when(jnp.logical_and(c == 0, s == NS - 1))
        def _():
            pltpu.async_copy(x_hbm.at[pl.ds((NS - 1) * RPT, XR)],
                             acc.at[pl.ds((NS - 1) * RPT, XR)], si0)
            pltpu.sync_copy(zeros_hbm.at[pl.ds(N, NP - N)],
                            acc.at[pl.ds(N, NP - N)])
        pltpu.sync_copy(src_hbm.at[pl.ds(w * EW, EW)], src_v)
        pltpu.sync_copy(dst2_hbm.at[pl.ds(r0, PH)], dstb0)
        pltpu.async_copy(dst2_hbm.at[pl.ds(r0 + PH, PH)], dstb1, si1)
        pltpu.async_copy(x_hbm.at[src_v.at[pl.ds(0, EB)]], rows0, sg0)
        pltpu.async_copy(x_hbm.at[src_v.at[pl.ds(EB, EB)]], rows1, sg1)
        pltpu.async_copy(x_hbm.at[src_v.at[pl.ds(2 * EB, EB)]], rows2, sg2)
        @pl.when(jnp.logical_or(c == 1, s < NS - 1))
        def _():
            pltpu.make_async_copy(zeros_hbm.at[pl.ds(s * RPT, RPT)],
                                  acc.at[pl.ds(s * RPT, RPT)], si0).wait()

        @pl.when(jnp.logical_and(c == 0, s == NS - 1))
        def _():
            pltpu.make_async_copy(x_hbm.at[pl.ds((NS - 1) * RPT, XR)],
                                  acc.at[pl.ds((NS - 1) * RPT, XR)],
                                  si0).wait()
        plsc.subcore_barrier()

        def step(b_dyn, j, dstb, jj):
            k = j & 3
            pltpu.make_async_copy(
                x_hbm.at[src_v.at[pl.ds(0, EB)]], rows[k], sg[k]).wait()
            nxt = (b_dyn + (j + 3)) * EB
            kn = (j + 3) & 3

            def start_next():
                pltpu.async_copy(
                    x_hbm.at[src_v.at[pl.ds(nxt, EB)]], rows[kn], sg[kn])
            if j < 13:
                start_next()
            else:
                pl.when(b_dyn + j + 3 < NB)(start_next)
            pltpu.sync_copy(rows[k], acc.at[dstb.at[jj]], add=True)

        def body(i, carry):
            b_dyn = i * 2 * PH

            @pl.when(i > 0)
            def _():
                pltpu.make_async_copy(
                    dst2_hbm.at[pl.ds(r0, PH)], dstb0, si0).wait()
            for j in range(PH):
                step(b_dyn, j, dstb0, j)
            pltpu.make_async_copy(
                dst2_hbm.at[pl.ds(r0, PH)], dstb1, si1).wait()

            @pl.when(i < NPH // 2 - 1)
            def _():
                pltpu.async_copy(
                    dst2_hbm.at[pl.ds(r0 + (i * 2 + 2) * PH, PH)], dstb0, si0)
            for j in range(PH):
                step(b_dyn, PH + j, dstb1, j)

            @pl.when(i < NPH // 2 - 1)
            def _():
                pltpu.async_copy(
                    dst2_hbm.at[pl.ds(r0 + (i * 2 + 3) * PH, PH)], dstb1, si1)
            return carry

        lax.fori_loop(0, NPH // 2, body, 0, unroll=False)

        plsc.subcore_barrier()
        pltpu.sync_copy(acc.at[pl.ds(s * RPT, RPT)],
                        out_hbm.at[pl.ds(c * NP + s * RPT, RPT)])

    return sc_agg


_sc_agg = _sc_agg_build()


def _mlp_kernel(relu_out, p0_ref, p1_ref, wa_ref, ba_ref, wb_ref,
                bb_ref, o_ref):
    rst = p0_ref[...] + p1_ref[...]
    hid = jnp.dot(rst, wa_ref[...], preferred_element_type=jnp.float32)
    hid = jnp.maximum(hid + ba_ref[...], 0.0)
    out = jnp.dot(hid, wb_ref[...], preferred_element_type=jnp.float32)
    out = out + bb_ref[...]
    if relu_out:
        out = jnp.maximum(out, 0.0)
    o_ref[...] = out


def _mlp(p0, p1, Wa, ba, Wb, bb, relu_out, bn=2000):
    grid = N // bn
    return pl.pallas_call(
        functools.partial(_mlp_kernel, relu_out),
        grid=(grid,),
        in_specs=[
            pl.BlockSpec((bn, D), lambda i: (i, 0)),
            pl.BlockSpec((bn, D), lambda i: (i, 0)),
            pl.BlockSpec((D, D), lambda i: (0, 0)),
            pl.BlockSpec((1, D), lambda i: (0, 0)),
            pl.BlockSpec((D, D), lambda i: (0, 0)),
            pl.BlockSpec((1, D), lambda i: (0, 0)),
        ],
        out_specs=pl.BlockSpec((bn, D), lambda i: (i, 0)),
        out_shape=jax.ShapeDtypeStruct((N, D), jnp.float32),
    )(p0, p1, Wa, ba.reshape(1, D), Wb, bb.reshape(1, D))


def kernel(h, edge_index, W0a, b0a, W0b, b0b, W1a, b1a, W1b, b1b):
    x = h.T
    pad = EP - E
    pad_ar = jnp.arange(pad, dtype=jnp.int32)
    src = jnp.concatenate([edge_index[0], pad_ar % N])
    dst = jnp.concatenate([edge_index[1], N + pad_ar % (NP - N)])
    dst2 = dst.reshape(EP // EB, EB)

    zeros = jnp.zeros((NP, D), jnp.float32)
    p = _sc_agg(x, src, dst2, zeros)
    x1 = _mlp(p[:N], p[NP:NP + N], W0a, b0a, W0b, b0b, relu_out=True)
    p2 = _sc_agg(x1, src, dst2, zeros)
    x2 = _mlp(p2[:N], p2[NP:NP + N], W1a, b1a, W1b, b1b, relu_out=False)
    return x2.T


# speedup vs baseline: 1.0652x; 1.0090x over previous
import functools

import jax
import jax.numpy as jnp
from jax import lax
from jax.experimental import pallas as pl
from jax.experimental.pallas import tpu as pltpu
from jax.experimental.pallas import tpu_sc as plsc

N = 10000
E = 320000
D = 128

NC = 2
NS = 16
NW = NC * NS

EB = 64
NB = 160
EW = NB * EB
EP = NW * EW
PH = 8
NPH = NB // PH

NP = 10112
RPT = NP // NS


def _sc_agg_build():
    mesh = plsc.VectorSubcoreMesh(core_axis_name="c", subcore_axis_name="s")

    @functools.partial(
        pl.kernel,
        out_type=jax.ShapeDtypeStruct((NC * NP, D), jnp.float32),
        mesh=mesh,
        scratch_types=[
            pltpu.VMEM((EW,), jnp.int32),
            pltpu.VMEM((PH, EB), jnp.int32),
            pltpu.VMEM((PH, EB), jnp.int32),
            pltpu.VMEM((EB, D), jnp.float32),
            pltpu.VMEM((EB, D), jnp.float32),
            pltpu.VMEM((EB, D), jnp.float32),
            pltpu.VMEM((EB, D), jnp.float32),
            pltpu.VMEM_SHARED((NP, D), jnp.float32),
            pltpu.SemaphoreType.DMA,
            pltpu.SemaphoreType.DMA,
            pltpu.SemaphoreType.DMA,
            pltpu.SemaphoreType.DMA,
            pltpu.SemaphoreType.DMA,
            pltpu.SemaphoreType.DMA,
        ],
    )
    def sc_agg(x_hbm, src_hbm, dst2_hbm, zeros_hbm, out_hbm,
               src_v, dstb0, dstb1, rows0, rows1, rows2, rows3, acc,
               sg0, sg1, sg2, sg3, si0, si1):
        c = lax.axis_index("c")
        s = lax.axis_index("s")
        w = c * NS + s
        r0 = w * NB

        rows = (rows0, rows1, rows2, rows3)
        sg = (sg0, sg1, sg2, sg3)

        XR = N - (NS - 1) * RPT

        @pl.when(c == 1)
        def _():
            pltpu.async_copy(zeros_hbm,
                             acc.at[pl.ds(s * RPT, RPT)], si0)

        @pl.when(jnp.logical_and(c == 0, s < NS - 1))
        def _():
            pltpu.async_copy(x_hbm.at[pl.ds(s * RPT, RPT)],
                             acc.at[pl.ds(s * RPT, RPT)], si0)

        @pl.when(jnp.logical_and(c == 0, s == NS - 1))
        def _():
            pltpu.async_copy(x_hbm.at[pl.ds((NS - 1) * RPT, XR)],
                             acc.at[pl.ds((NS - 1) * RPT, XR)], si0)
            pltpu.sync_copy(zeros_hbm.at[pl.ds(0, NP - N)],
                            acc.at[pl.ds(N, NP - N)])
        pltpu.sync_copy(src_hbm.at[pl.ds(w * EW, EW)], src_v)
        pltpu.sync_copy(dst2_hbm.at[pl.ds(r0, PH)], dstb0)
        pltpu.async_copy(dst2_hbm.at[pl.ds(r0 + PH, PH)], dstb1, si1)
        pltpu.async_copy(x_hbm.at[src_v.at[pl.ds(0, EB)]], rows0, sg0)
        pltpu.async_copy(x_hbm.at[src_v.at[pl.ds(EB, EB)]], rows1, sg1)
        pltpu.async_copy(x_hbm.at[src_v.at[pl.ds(2 * EB, EB)]], rows2, sg2)
        @pl.when(jnp.logical_or(c == 1, s < NS - 1))
        def _():
            pltpu.make_async_copy(zeros_hbm,
                                  acc.at[pl.ds(s * RPT, RPT)], si0).wait()

        @pl.when(jnp.logical_and(c == 0, s == NS - 1))
        def _():
            pltpu.make_async_copy(x_hbm.at[pl.ds((NS - 1) * RPT, XR)],
                                  acc.at[pl.ds((NS - 1) * RPT, XR)],
                                  si0).wait()
        plsc.subcore_barrier()

        def step(b_dyn, j, dstb, jj):
            k = j & 3
            pltpu.make_async_copy(
                x_hbm.at[src_v.at[pl.ds(0, EB)]], rows[k], sg[k]).wait()
            nxt = (b_dyn + (j + 3)) * EB
            kn = (j + 3) & 3

            def start_next():
                pltpu.async_copy(
                    x_hbm.at[src_v.at[pl.ds(nxt, EB)]], rows[kn], sg[kn])
            if j < 13:
                start_next()
            else:
                pl.when(b_dyn + j + 3 < NB)(start_next)
            pltpu.sync_copy(rows[k], acc.at[dstb.at[jj]], add=True)

        def body(i, carry):
            b_dyn = i * 2 * PH

            @pl.when(i > 0)
            def _():
                pltpu.make_async_copy(
                    dst2_hbm.at[pl.ds(r0, PH)], dstb0, si0).wait()
            for j in range(PH):
                step(b_dyn, j, dstb0, j)
            pltpu.make_async_copy(
                dst2_hbm.at[pl.ds(r0, PH)], dstb1, si1).wait()

            @pl.when(i < NPH // 2 - 1)
            def _():
                pltpu.async_copy(
                    dst2_hbm.at[pl.ds(r0 + (i * 2 + 2) * PH, PH)], dstb0, si0)
            for j in range(PH):
                step(b_dyn, PH + j, dstb1, j)

            @pl.when(i < NPH // 2 - 1)
            def _():
                pltpu.async_copy(
                    dst2_hbm.at[pl.ds(r0 + (i * 2 + 3) * PH, PH)], dstb1, si1)
            return carry

        lax.fori_loop(0, NPH // 2, body, 0, unroll=False)

        plsc.subcore_barrier()
        pltpu.sync_copy(acc.at[pl.ds(s * RPT, RPT)],
                        out_hbm.at[pl.ds(c * NP + s * RPT, RPT)])

    return sc_agg


_sc_agg = _sc_agg_build()


def _mlp_kernel(relu_out, p0_ref, p1_ref, wa_ref, ba_ref, wb_ref,
                bb_ref, o_ref):
    rst = p0_ref[...] + p1_ref[...]
    hid = jnp.dot(rst, wa_ref[...], preferred_element_type=jnp.float32)
    hid = jnp.maximum(hid + ba_ref[...], 0.0)
    out = jnp.dot(hid, wb_ref[...], preferred_element_type=jnp.float32)
    out = out + bb_ref[...]
    if relu_out:
        out = jnp.maximum(out, 0.0)
    o_ref[...] = out


def _mlp(p0, p1, Wa, ba, Wb, bb, relu_out, bn=2000):
    grid = N // bn
    return pl.pallas_call(
        functools.partial(_mlp_kernel, relu_out),
        grid=(grid,),
        in_specs=[
            pl.BlockSpec((bn, D), lambda i: (i, 0)),
            pl.BlockSpec((bn, D), lambda i: (i, 0)),
            pl.BlockSpec((D, D), lambda i: (0, 0)),
            pl.BlockSpec((1, D), lambda i: (0, 0)),
            pl.BlockSpec((D, D), lambda i: (0, 0)),
            pl.BlockSpec((1, D), lambda i: (0, 0)),
        ],
        out_specs=pl.BlockSpec((bn, D), lambda i: (i, 0)),
        out_shape=jax.ShapeDtypeStruct((N, D), jnp.float32),
    )(p0, p1, Wa, ba.reshape(1, D), Wb, bb.reshape(1, D))


def kernel(h, edge_index, W0a, b0a, W0b, b0b, W1a, b1a, W1b, b1b):
    x = h.T
    pad = EP - E
    pad_ar = jnp.arange(pad, dtype=jnp.int32)
    src = jnp.concatenate([edge_index[0], pad_ar % N])
    dst = jnp.concatenate([edge_index[1], N + pad_ar % (NP - N)])
    dst2 = dst.reshape(EP // EB, EB)

    zeros = jnp.zeros((RPT, D), jnp.float32)
    p = _sc_agg(x, src, dst2, zeros)
    x1 = _mlp(p[:N], p[NP:NP + N], W0a, b0a, W0b, b0b, relu_out=True)
    p2 = _sc_agg(x1, src, dst2, zeros)
    x2 = _mlp(p2[:N], p2[NP:NP + N], W1a, b1a, W1b, b1b, relu_out=False)
    return x2.T
